# Initial kernel scaffold; baseline (speedup 1.0000x reference)
import jax
import jax.numpy as jnp
from jax.experimental import pallas as pl


def kernel(x, edge_index, W1, att_src1, att_dst1, b1, W2, att_src2, att_dst2, b2):
    raise NotImplementedError("write your pallas kernel here")


# speedup vs baseline: 70.7740x; 70.7740x over previous
import jax
import jax.numpy as jnp
from jax import lax
from jax.experimental import pallas as pl
from jax.experimental.pallas import tpu as pltpu
from jax.experimental.pallas import tpu_sc as plsc

N = 10000
E = 320000
D_IN = 128
H1 = 64
C2 = 16
W1ROW = H1 + 8
W2ROW = 2 * C2
NEG = 0.2

NP = 10240
RPT = NP // 16
CH = 128
NCHUNK = 82
EPT = NCHUNK * CH
ETOT = 32 * EPT
ROWBLK = 1024
HIGHEST = jax.lax.Precision.HIGHEST


def _tc1_body(x_ref, w1_ref, asrc_ref, adst_ref, xp_ref, as_ref, ad_ref):
    xp = jnp.dot(x_ref[...], w1_ref[...], preferred_element_type=jnp.float32,
                 precision=HIGHEST)
    xp_ref[...] = xp
    as_ref[...] = jnp.dot(xp, asrc_ref[...], preferred_element_type=jnp.float32,
                          precision=HIGHEST)
    ad_ref[...] = jnp.dot(xp, adst_ref[...], preferred_element_type=jnp.float32,
                          precision=HIGHEST)


def _tc1(xpad, W1, Asrc, Adst):
    return pl.pallas_call(
        _tc1_body,
        grid=(NP // ROWBLK,),
        in_specs=[
            pl.BlockSpec((ROWBLK, D_IN), lambda i: (i, 0)),
            pl.BlockSpec((D_IN, H1), lambda i: (0, 0)),
            pl.BlockSpec((H1, 8), lambda i: (0, 0)),
            pl.BlockSpec((H1, 8), lambda i: (0, 0)),
        ],
        out_specs=[
            pl.BlockSpec((ROWBLK, H1), lambda i: (i, 0)),
            pl.BlockSpec((ROWBLK, 8), lambda i: (i, 0)),
            pl.BlockSpec((ROWBLK, 8), lambda i: (i, 0)),
        ],
        out_shape=[
            jax.ShapeDtypeStruct((NP, H1), jnp.float32),
            jax.ShapeDtypeStruct((NP, 8), jnp.float32),
            jax.ShapeDtypeStruct((NP, 8), jnp.float32),
        ],
    )(xpad, W1, Asrc, Adst)


def _tc2_body(outp_ref, b1_ref, w2_ref, bmat_ref, a2s_ref, a2d_ref,
              xp2_ref, as2_ref, ad2_ref):
    o = outp_ref[...]
    acc = o[0] + o[1]
    num = acc[:, :H1]
    ssum = acc[:, H1:H1 + 8]
    s64 = jnp.dot(ssum, bmat_ref[...], preferred_element_type=jnp.float32,
                  precision=HIGHEST)
    h = num / (s64 + 1e-16) + b1_ref[...]
    h = jnp.where(h > 0, h, jnp.exp(h) - 1.0)
    xp2 = jnp.dot(h, w2_ref[...], preferred_element_type=jnp.float32,
                  precision=HIGHEST)
    xp2_ref[...] = xp2
    as2_ref[...] = jnp.dot(xp2, a2s_ref[...], preferred_element_type=jnp.float32,
                           precision=HIGHEST)
    ad2_ref[...] = jnp.dot(xp2, a2d_ref[...], preferred_element_type=jnp.float32,
                           precision=HIGHEST)


def _tc2(outp, b1r, W2, Bmat, a2s, a2d):
    return pl.pallas_call(
        _tc2_body,
        grid=(NP // ROWBLK,),
        in_specs=[
            pl.BlockSpec((2, ROWBLK, W1ROW), lambda i: (0, i, 0)),
            pl.BlockSpec((1, H1), lambda i: (0, 0)),
            pl.BlockSpec((H1, C2), lambda i: (0, 0)),
            pl.BlockSpec((8, H1), lambda i: (0, 0)),
            pl.BlockSpec((C2, 8), lambda i: (0, 0)),
            pl.BlockSpec((C2, 8), lambda i: (0, 0)),
        ],
        out_specs=[
            pl.BlockSpec((ROWBLK, C2), lambda i: (i, 0)),
            pl.BlockSpec((ROWBLK, 8), lambda i: (i, 0)),
            pl.BlockSpec((ROWBLK, 8), lambda i: (i, 0)),
        ],
        out_shape=[
            jax.ShapeDtypeStruct((NP, C2), jnp.float32),
            jax.ShapeDtypeStruct((NP, 8), jnp.float32),
            jax.ShapeDtypeStruct((NP, 8), jnp.float32),
        ],
    )(outp, b1r, W2, Bmat, a2s, a2d)


def _tc3_body(out2p_ref, b2_ref, o_ref):
    o = out2p_ref[...]
    acc = o[0] + o[1]
    num = acc[:, :C2]
    ssum = acc[:, C2:C2 + 1]
    o_ref[...] = num / (ssum + 1e-16) + b2_ref[...]


def _tc3(out2p, b2r):
    return pl.pallas_call(
        _tc3_body,
        grid=(NP // ROWBLK,),
        in_specs=[
            pl.BlockSpec((2, ROWBLK, W2ROW), lambda i: (0, i, 0)),
            pl.BlockSpec((1, C2), lambda i: (0, 0)),
        ],
        out_specs=pl.BlockSpec((ROWBLK, C2), lambda i: (i, 0)),
        out_shape=jax.ShapeDtypeStruct((NP, C2), jnp.float32),
    )(out2p, b2r)


_SC_PARAMS = pltpu.CompilerParams(use_tc_tiling_on_sc=False,
                                  needs_layout_passes=False)


def _sc1_body(src_hbm, dst_hbm, xp_hbm, as_hbm, ad_hbm, m_hbm,
              outp_hbm,
              src_i, dst_i, as_b, ad_b, xp_b, e_b, msg_b, m_v,
              out_sh, gsem0, gsem1, ssem0, ssem1):
    cid = lax.axis_index("c")
    sid = lax.axis_index("s")
    wid = cid * 16 + sid
    lane = lax.iota(jnp.int32, 16)
    rowpat = lax.shift_right_logical(lane, 3)
    colpat = lane & 7
    zero16 = jnp.zeros((16,), jnp.float32)
    gsem = (gsem0, gsem1)
    ssem = (ssem0, ssem1)

    @pl.loop(0, CH)
    def _zero_msg(k):
        for j in range(4):
            msg_b[0, k, pl.ds(j * 16, 16)] = zero16

    @pl.loop(0, 64)
    def _zero_tail(i):
        plsc.store_scatter(msg_b.at[0], [2 * i + rowpat, H1 + colpat], zero16)

    @pl.loop(0, RPT // CH)
    def _zero_sh(t):
        pltpu.sync_copy(msg_b.at[0], out_sh.at[pl.ds(sid * RPT + t * CH, CH)])

    pltpu.sync_copy(m_hbm, m_v)
    mv = m_v[...]
    plsc.subcore_barrier()

    def fire(b, i):
        base = wid * EPT + i * CH
        pltpu.sync_copy(src_hbm.at[pl.ds(base, CH)], src_i.at[b])
        pltpu.sync_copy(dst_hbm.at[pl.ds(base, CH)], dst_i.at[b])
        pltpu.async_copy(as_hbm.at[src_i.at[b]], as_b.at[b], gsem[b])
        pltpu.async_copy(ad_hbm.at[dst_i.at[b]], ad_b.at[b], gsem[b])
        pltpu.async_copy(xp_hbm.at[src_i.at[b]], xp_b.at[b], gsem[b])

    fire(0, 0)

    @pl.loop(0, NCHUNK, step=2)
    def _pair(g):
        for b in range(2):
            ob = 1 - b
            gi = g + b
            pltpu.make_async_copy(as_hbm.at[src_i.at[b]], as_b.at[b], gsem[b]).wait()
            pltpu.make_async_copy(ad_hbm.at[dst_i.at[b]], ad_b.at[b], gsem[b]).wait()
            pltpu.make_async_copy(xp_hbm.at[src_i.at[b]], xp_b.at[b], gsem[b]).wait()

            @pl.when(gi >= 1)
            def _():
                pltpu.make_async_copy(msg_b.at[ob], out_sh.at[dst_i.at[ob]],
                                      ssem[ob]).wait()

            @pl.when(gi + 1 < NCHUNK)
            def _():
                fire(ob, gi + 1)

            @pl.loop(0, 64, unroll=2)
            def _alpha(i):
                row = 2 * i + rowpat
                va = plsc.load_gather(as_b.at[b], [row, colpat])
                vd = plsc.load_gather(ad_b.at[b], [row, colpat])
                al = va + vd
                al = jnp.where(al > 0, al, al * NEG)
                ev = jnp.exp(al - mv)
                plsc.store_scatter(e_b, [row, colpat], ev)
                plsc.store_scatter(msg_b.at[b], [row, H1 + colpat], ev)

            @pl.loop(0, CH, unroll=4)
            def _msg(k):
                krow = jnp.broadcast_to(k, (16,))
                for j in range(4):
                    evj = plsc.load_gather(e_b, [krow, rowpat + 2 * j])
                    msg_b[b, k, pl.ds(j * 16, 16)] = \
                        evj * xp_b[b, k, pl.ds(j * 16, 16)]

            pltpu.async_copy(msg_b.at[b], out_sh.at[dst_i.at[b]], ssem[b],
                             add=True)

    lastb = (NCHUNK - 1) % 2
    pltpu.make_async_copy(msg_b.at[lastb], out_sh.at[dst_i.at[lastb]],
                          ssem[lastb]).wait()
    plsc.subcore_barrier()
    rbase = sid * RPT
    pltpu.sync_copy(out_sh.at[pl.ds(rbase, RPT)],
                    outp_hbm.at[cid, pl.ds(rbase, RPT)])


def _sc1(src, dst, xp1, as1, ad1, m1v):
    mesh = plsc.VectorSubcoreMesh(core_axis_name="c", subcore_axis_name="s")
    f = pl.kernel(
        _sc1_body,
        out_type=[
            jax.ShapeDtypeStruct((2, NP, W1ROW), jnp.float32),
        ],
        mesh=mesh,
        compiler_params=_SC_PARAMS,
        scratch_types=[
            pltpu.VMEM((2, CH), jnp.int32),
            pltpu.VMEM((2, CH), jnp.int32),
            pltpu.VMEM((2, CH, 8), jnp.float32),
            pltpu.VMEM((2, CH, 8), jnp.float32),
            pltpu.VMEM((2, CH, H1), jnp.float32),
            pltpu.VMEM((CH, 8), jnp.float32),
            pltpu.VMEM((2, CH, W1ROW), jnp.float32),
            pltpu.VMEM((16,), jnp.float32),
            pltpu.VMEM_SHARED((NP, W1ROW), jnp.float32),
            pltpu.SemaphoreType.DMA,
            pltpu.SemaphoreType.DMA,
            pltpu.SemaphoreType.DMA,
            pltpu.SemaphoreType.DMA,
        ],
    )
    return f(src, dst, xp1, as1, ad1, m1v)[0]


def _sc2_body(src_hbm, dst_hbm, xp2_hbm, as2_hbm, ad2_hbm, m2_hbm,
              out2p_hbm,
              src_i, dst_i, as2_v, ad2_v, xp2_b, e_b2, msg2_b, m2_v,
              out2_sh, gsem0, gsem1, ssem0, ssem1):
    cid = lax.axis_index("c")
    sid = lax.axis_index("s")
    wid = cid * 16 + sid
    lane = lax.iota(jnp.int32, 16)
    izero = jnp.zeros((16,), jnp.int32)
    zero16 = jnp.zeros((16,), jnp.float32)
    gsem = (gsem0, gsem1)
    ssem = (ssem0, ssem1)

    @pl.loop(0, CH)
    def _zero_msg(k):
        msg2_b[0, k, pl.ds(0, 16)] = zero16
        msg2_b[0, k, pl.ds(16, 16)] = zero16
        msg2_b[1, k, pl.ds(16, 16)] = zero16

    @pl.loop(0, RPT // CH)
    def _zero_sh(t):
        pltpu.sync_copy(msg2_b.at[0], out2_sh.at[pl.ds(sid * RPT + t * CH, CH)])

    pltpu.sync_copy(as2_hbm, as2_v)
    pltpu.sync_copy(ad2_hbm, ad2_v)
    pltpu.sync_copy(m2_hbm, m2_v)
    mv = m2_v[...]
    plsc.subcore_barrier()

    def fire(b, i):
        base = wid * EPT + i * CH
        pltpu.sync_copy(src_hbm.at[pl.ds(base, CH)], src_i.at[b])
        pltpu.sync_copy(dst_hbm.at[pl.ds(base, CH)], dst_i.at[b])
        pltpu.async_copy(xp2_hbm.at[src_i.at[b]], xp2_b.at[b], gsem[b])

    fire(0, 0)

    @pl.loop(0, NCHUNK, step=2)
    def _pair(g):
        for b in range(2):
            ob = 1 - b
            gi = g + b
            pltpu.make_async_copy(xp2_hbm.at[src_i.at[b]], xp2_b.at[b],
                                  gsem[b]).wait()

            @pl.when(gi >= 1)
            def _():
                pltpu.make_async_copy(msg2_b.at[ob], out2_sh.at[dst_i.at[ob]],
                                      ssem[ob]).wait()

            @pl.when(gi + 1 < NCHUNK)
            def _():
                fire(ob, gi + 1)

            @pl.loop(0, 8)
            def _alpha(i):
                sv = src_i[b, pl.ds(i * 16, 16)]
                dv = dst_i[b, pl.ds(i * 16, 16)]
                va = plsc.load_gather(as2_v, [sv])
                vd = plsc.load_gather(ad2_v, [dv])
                al = va + vd
                al = jnp.where(al > 0, al, al * NEG)
                plsc.store_scatter(e_b2, [i * 16 + lane, izero],
                                   jnp.exp(al - mv))

            @pl.loop(0, CH, unroll=4)
            def _msg(k):
                krow = jnp.broadcast_to(k, (16,))
                ev = plsc.load_gather(e_b2, [krow, izero])
                msg2_b[b, k, pl.ds(0, 16)] = ev * xp2_b[b, k, pl.ds(0, 16)]
                msg2_b[b, k, pl.ds(16, 16)] = jnp.where(lane == 0, ev, 0.0)

            pltpu.async_copy(msg2_b.at[b], out2_sh.at[dst_i.at[b]], ssem[b],
                             add=True)

    lastb = (NCHUNK - 1) % 2
    pltpu.make_async_copy(msg2_b.at[lastb], out2_sh.at[dst_i.at[lastb]],
                          ssem[lastb]).wait()
    plsc.subcore_barrier()
    rbase = sid * RPT
    pltpu.sync_copy(out2_sh.at[pl.ds(rbase, RPT)],
                    out2p_hbm.at[cid, pl.ds(rbase, RPT)])


def _sc2(src, dst, xp2, as2, ad2, m2v):
    mesh = plsc.VectorSubcoreMesh(core_axis_name="c", subcore_axis_name="s")
    f = pl.kernel(
        _sc2_body,
        out_type=[
            jax.ShapeDtypeStruct((2, NP, W2ROW), jnp.float32),
        ],
        mesh=mesh,
        compiler_params=_SC_PARAMS,
        scratch_types=[
            pltpu.VMEM((2, CH), jnp.int32),
            pltpu.VMEM((2, CH), jnp.int32),
            pltpu.VMEM((NP,), jnp.float32),
            pltpu.VMEM((NP,), jnp.float32),
            pltpu.VMEM((2, CH, C2), jnp.float32),
            pltpu.VMEM((CH, 1), jnp.float32),
            pltpu.VMEM((2, CH, W2ROW), jnp.float32),
            pltpu.VMEM((16,), jnp.float32),
            pltpu.VMEM_SHARED((NP, W2ROW), jnp.float32),
            pltpu.SemaphoreType.DMA,
            pltpu.SemaphoreType.DMA,
            pltpu.SemaphoreType.DMA,
            pltpu.SemaphoreType.DMA,
        ],
    )
    return f(src, dst, xp2, as2, ad2, m2v)[0]


def kernel(x, edge_index, W1, att_src1, att_dst1, b1, W2, att_src2, att_dst2, b2):
    f32 = jnp.float32
    pad = ETOT - E - N
    loop = jnp.arange(N, dtype=jnp.int32)
    dummy = jnp.full((pad,), N, jnp.int32)
    src = jnp.concatenate([edge_index[0], loop, dummy])
    dst = jnp.concatenate([edge_index[1], loop, dummy])

    xpad = jnp.pad(x, ((0, NP - N), (0, 0)))
    eye8 = jnp.eye(8, dtype=f32)
    Asrc = jnp.einsum("hc,hg->hcg", att_src1[0], eye8).reshape(H1, 8)
    Adst = jnp.einsum("hc,hg->hcg", att_dst1[0], eye8).reshape(H1, 8)

    xp1, as1, ad1 = _tc1(xpad, W1, Asrc, Adst)
    m1 = jax.nn.relu(jnp.max(as1, axis=0) + jnp.max(ad1, axis=0))
    m1v = jnp.tile(m1, 2)

    outp = _sc1(src, dst, xp1, as1, ad1, m1v)

    Bmat = jnp.kron(eye8, jnp.ones((1, 8), f32))
    a2s = jnp.zeros((C2, 8), f32).at[:, 0].set(att_src2.reshape(C2))
    a2d = jnp.zeros((C2, 8), f32).at[:, 0].set(att_dst2.reshape(C2))
    xp2, as2o, ad2o = _tc2(outp, b1.reshape(1, H1), W2, Bmat, a2s, a2d)
    as2 = as2o[:, 0]
    ad2 = ad2o[:, 0]
    m2 = jax.nn.relu(jnp.max(as2) + jnp.max(ad2))
    m2v = jnp.full((16,), m2, f32)

    out2p = _sc2(src, dst, xp2, as2, ad2, m2v)
    out = _tc3(out2p, b2.reshape(1, C2))
    return out[:N]
